# baseline (speedup 1.0000x reference)
import functools

import jax
import jax.numpy as jnp
from jax import lax
from jax.experimental import pallas as pl
from jax.experimental.pallas import tpu as pltpu
from jax.experimental.pallas import tpu_sc as plsc

BATCH = 16384
DIM = 64
NUM_ROWS = 1000000
NUM_WORKERS = 32
B_PER_W = BATCH // NUM_WORKERS
CHUNK = 128
CHUNKS_PER_W = B_PER_W // CHUNK
SLICES_PER_CHUNK = CHUNK // 16


def _sc_body(uidx_hbm, iidx_hbm, uf_hbm, vf_hbm, ub_hbm, vb_hbm, gb_hbm,
             out_hbm, idx_u, idx_i, flat_u, flat_i, uf_v, vf_v,
             ub_v, vb_v, out_v, gb_v, sem, bsem):
    wid = lax.axis_index("s") * 2 + lax.axis_index("c")
    chunk_base = wid * CHUNKS_PER_W

    pltpu.sync_copy(uidx_hbm.at[pl.ds(chunk_base, CHUNKS_PER_W), :], idx_u)
    pltpu.sync_copy(iidx_hbm.at[pl.ds(chunk_base, CHUNKS_PER_W), :], idx_i)
    pltpu.sync_copy(gb_hbm, gb_v)

    bias_copies = []
    for ch in range(CHUNKS_PER_W):
        rows_sl = pl.ds(ch * CHUNK, CHUNK)
        bias_copies.append(pltpu.async_copy(ub_hbm.at[idx_u.at[ch]],
                                            ub_v.at[rows_sl], bsem))
        bias_copies.append(pltpu.async_copy(vb_hbm.at[idx_i.at[ch]],
                                            vb_v.at[rows_sl], bsem))

    lanes = lax.iota(jnp.int32, 16)
    gb16 = gb_v[...]

    for ch in range(CHUNKS_PER_W):
        def build_body(d, carry, _ch=ch):
            for s in range(SLICES_PER_CHUNK):
                sl = pl.ds(s * 16, 16)
                r_u = idx_u[_ch, pl.ds(s * 16, 16)]
                r_i = idx_i[_ch, pl.ds(s * 16, 16)]
                flat_u[d, sl] = r_u + d * NUM_ROWS
                flat_i[d, sl] = r_i + d * NUM_ROWS
            return carry

        lax.fori_loop(0, DIM, build_body, 0)

        copies = []
        for d in range(DIM):
            copies.append(pltpu.async_copy(uf_hbm.at[flat_u.at[d]],
                                           uf_v.at[d], sem))
            copies.append(pltpu.async_copy(vf_hbm.at[flat_i.at[d]],
                                           vf_v.at[d], sem))
        for c in copies:
            c.wait()

        def slice_body(s, carry, _ch=ch):
            acc = jnp.zeros((16,), jnp.float32)
            sl = pl.ds(s * 16, 16)
            for d in range(DIM):
                acc = acc + uf_v[d, pl.ds(s * 16, 16)] * vf_v[d, pl.ds(s * 16, 16)]
            base = _ch * CHUNK
            ub16 = ub_v[pl.ds(base + s * 16, 16)]
            vb16 = vb_v[pl.ds(base + s * 16, 16)]
            out_v[pl.ds(base + s * 16, 16)] = acc + ub16 + vb16 + gb16
            return carry

        lax.fori_loop(0, SLICES_PER_CHUNK, slice_body, 0)

    for c in bias_copies:
        c.wait()
    pltpu.sync_copy(out_v, out_hbm.at[pl.ds(wid * B_PER_W, B_PER_W)])


@functools.partial(
    pl.kernel,
    out_type=jax.ShapeDtypeStruct((BATCH,), jnp.float32),
    mesh=plsc.VectorSubcoreMesh(core_axis_name="c", subcore_axis_name="s"),
    scratch_types=[
        pltpu.VMEM((CHUNKS_PER_W, CHUNK), jnp.int32),
        pltpu.VMEM((CHUNKS_PER_W, CHUNK), jnp.int32),
        pltpu.VMEM((DIM, CHUNK), jnp.int32),
        pltpu.VMEM((DIM, CHUNK), jnp.int32),
        pltpu.VMEM((DIM, CHUNK), jnp.float32),
        pltpu.VMEM((DIM, CHUNK), jnp.float32),
        pltpu.VMEM((B_PER_W,), jnp.float32),
        pltpu.VMEM((B_PER_W,), jnp.float32),
        pltpu.VMEM((B_PER_W,), jnp.float32),
        pltpu.VMEM((16,), jnp.float32),
        pltpu.SemaphoreType.DMA,
        pltpu.SemaphoreType.DMA,
    ],
    compiler_params=pltpu.CompilerParams(needs_layout_passes=False,
                                         use_tc_tiling_on_sc=False),
)
def _sc_kernel(*refs):
    _sc_body(*refs)


def kernel(user_idx, item_idx, user_factors, item_factors, user_bias,
           item_bias, global_bias):
    uidx = user_idx.astype(jnp.int32).reshape(BATCH // CHUNK, CHUNK)
    iidx = item_idx.astype(jnp.int32).reshape(BATCH // CHUNK, CHUNK)
    uf_flat = user_factors.T.reshape(-1)
    vf_flat = item_factors.T.reshape(-1)
    gb = jnp.broadcast_to(global_bias, (16,))
    return _sc_kernel(uidx, iidx, uf_flat, vf_flat,
                      user_bias.reshape(-1), item_bias.reshape(-1), gb)


# speedup vs baseline: 20.9060x; 20.9060x over previous
import functools

import jax
import jax.numpy as jnp
from jax import lax
from jax.experimental import pallas as pl
from jax.experimental.pallas import tpu as pltpu
from jax.experimental.pallas import tpu_sc as plsc

BATCH = 16384
DIM = 64
PAD_DIM = 128
NUM_ROWS = 1000000
NUM_WORKERS = 32
B_PER_W = BATCH // NUM_WORKERS
CHUNK = 128
CHUNKS_PER_W = B_PER_W // CHUNK
GROUPS_PER_CHUNK = CHUNK // 16
PANEL = 512
FULL_PANELS = NUM_ROWS // PANEL
TAIL_START = FULL_PANELS * PANEL
TAIL = NUM_ROWS - TAIL_START


def _sweep_table(tab_hbm, tail_hbm, r_v, out_hbm, out_base, panel_a, panel_b,
                 tail_v, st2, sem):
    pltpu.sync_copy(tail_hbm, tail_v)
    lanes = lax.iota(jnp.int32, 16)
    first = plsc.load_gather(r_v, [jnp.zeros((16,), jnp.int32), lanes])
    k0 = lax.reduce_min(first, axes=(0,)) // PANEL

    for f in range(CHUNKS_PER_W):
        def group_body(g, k_next, _f=f):
            r16 = r_v[_f, pl.ds(g * 16, 16)]
            j16 = g * 16 + lanes

            def n_undone(done):
                cnt = plsc.all_reduce_population_count(~done)
                return lax.reduce_max(cnt, axes=(0,))

            def cond(carry):
                _, done = carry
                return n_undone(done) > 0

            def body(carry):
                k, done = carry
                is_tail = r16 >= TAIL_START
                m = jnp.logical_and(
                    ~done, jnp.logical_or(r16 < k * PANEL, is_tail))
                odd = jnp.bitwise_and(r16 // PANEL, 1) == 1
                col = jnp.bitwise_and(r16, PANEL - 1)
                col_t = jnp.maximum(r16 - TAIL_START, 0)
                for d in range(DIM):
                    d16 = jnp.full((16,), d, jnp.int32)
                    v_a = plsc.load_gather(panel_a, [d16, col])
                    v_b = plsc.load_gather(panel_b, [d16, col])
                    v_tail = plsc.load_gather(tail_v, [d16, col_t])
                    vals = jnp.where(is_tail, v_tail,
                                     jnp.where(odd, v_b, v_a))
                    plsc.store_scatter(st2, [j16, d16], vals, mask=m)
                done2 = jnp.logical_or(done, m)
                need_more = n_undone(done2) > 0

                @pl.when(need_more)
                def _():
                    kc = jnp.minimum(k, FULL_PANELS - 1)
                    src = tab_hbm.at[:, pl.ds(kc * PANEL, PANEL)]

                    @pl.when(jnp.bitwise_and(kc, 1) == 0)
                    def _():
                        pltpu.async_copy(src, panel_a, sem).wait()

                    @pl.when(jnp.bitwise_and(kc, 1) == 1)
                    def _():
                        pltpu.async_copy(src, panel_b, sem).wait()
                return (jnp.where(need_more, k + 1, k), done2)

            k_out, _ = lax.while_loop(
                cond, body, (k_next, jnp.zeros((16,), jnp.bool_)))
            return k_out

        k0 = lax.fori_loop(0, GROUPS_PER_CHUNK, group_body, k0)
        pltpu.sync_copy(st2, out_hbm.at[pl.ds(out_base + f * CHUNK, CHUNK), :])


def _k1_body(su_r_hbm, si_r_hbm, uf_hbm, vf_hbm,
             tail_u_hbm, tail_v_hbm, u_stage_hbm, v_stage_hbm,
             r_v, panel_a, panel_b, tail_v, st2, sem):
    wid = lax.axis_index("s") * 2 + lax.axis_index("c")
    chunk_base = wid * CHUNKS_PER_W
    out_base = wid * B_PER_W

    pltpu.sync_copy(su_r_hbm.at[pl.ds(chunk_base, CHUNKS_PER_W), :], r_v)
    _sweep_table(uf_hbm, tail_u_hbm, r_v, u_stage_hbm, out_base, panel_a,
                 panel_b, tail_v, st2, sem)

    pltpu.sync_copy(si_r_hbm.at[pl.ds(chunk_base, CHUNKS_PER_W), :], r_v)
    _sweep_table(vf_hbm, tail_v_hbm, r_v, v_stage_hbm, out_base, panel_a,
                 panel_b, tail_v, st2, sem)


@functools.partial(
    pl.kernel,
    out_type=(jax.ShapeDtypeStruct((BATCH, PAD_DIM), jnp.float32),
              jax.ShapeDtypeStruct((BATCH, PAD_DIM), jnp.float32)),
    mesh=plsc.VectorSubcoreMesh(core_axis_name="c", subcore_axis_name="s"),
    scratch_types=[
        pltpu.VMEM((CHUNKS_PER_W, CHUNK), jnp.int32),
        pltpu.VMEM((DIM, PANEL), jnp.float32),
        pltpu.VMEM((DIM, PANEL), jnp.float32),
        pltpu.VMEM((DIM, TAIL), jnp.float32),
        pltpu.VMEM((CHUNK, PAD_DIM), jnp.float32),
        pltpu.SemaphoreType.DMA,
    ],
    compiler_params=pltpu.CompilerParams(needs_layout_passes=False,
                                         use_tc_tiling_on_sc=True),
)
def _k1(*refs):
    _k1_body(*refs)


def _k2_body(uidx_hbm, iidx_hbm, inv_u_hbm, inv_i_hbm, u_stage_hbm,
             v_stage_hbm, ub_hbm, vb_hbm, gb_hbm, out_hbm, idx_u, idx_i,
             inv_u, inv_i, uf_v, vf_v, ub_v, vb_v, out_v, gb_v, sem, bsem):
    wid = lax.axis_index("s") * 2 + lax.axis_index("c")
    chunk_base = wid * CHUNKS_PER_W

    pltpu.sync_copy(uidx_hbm.at[pl.ds(chunk_base, CHUNKS_PER_W), :], idx_u)
    pltpu.sync_copy(iidx_hbm.at[pl.ds(chunk_base, CHUNKS_PER_W), :], idx_i)
    pltpu.sync_copy(inv_u_hbm.at[pl.ds(chunk_base, CHUNKS_PER_W), :], inv_u)
    pltpu.sync_copy(inv_i_hbm.at[pl.ds(chunk_base, CHUNKS_PER_W), :], inv_i)
    pltpu.sync_copy(gb_hbm, gb_v)

    bias_copies = []
    for ch in range(CHUNKS_PER_W):
        rows_sl = pl.ds(ch * CHUNK, CHUNK)
        bias_copies.append(pltpu.async_copy(ub_hbm.at[idx_u.at[ch]],
                                            ub_v.at[rows_sl], bsem))
        bias_copies.append(pltpu.async_copy(vb_hbm.at[idx_i.at[ch]],
                                            vb_v.at[rows_sl], bsem))
    for c in bias_copies:
        c.wait()

    lanes = lax.iota(jnp.int32, 16)
    gb16 = gb_v[...]

    for ch in range(CHUNKS_PER_W):
        cu = pltpu.async_copy(u_stage_hbm.at[inv_u.at[ch]], uf_v, sem)
        cv = pltpu.async_copy(v_stage_hbm.at[inv_i.at[ch]], vf_v, sem)
        cu.wait()
        cv.wait()

        def group_body(g, carry, _ch=ch):
            rows = g * 16 + lanes
            acc = jnp.zeros((16,), jnp.float32)
            for d in range(DIM):
                cols = jnp.full((16,), d, jnp.int32)
                acc = acc + (plsc.load_gather(uf_v, [rows, cols]) *
                             plsc.load_gather(vf_v, [rows, cols]))
            base = _ch * CHUNK
            ub16 = ub_v[pl.ds(base + g * 16, 16)]
            vb16 = vb_v[pl.ds(base + g * 16, 16)]
            out_v[pl.ds(base + g * 16, 16)] = acc + ub16 + vb16 + gb16
            return carry

        lax.fori_loop(0, GROUPS_PER_CHUNK, group_body, 0)

    pltpu.sync_copy(out_v, out_hbm.at[pl.ds(wid * B_PER_W, B_PER_W)])


@functools.partial(
    pl.kernel,
    out_type=jax.ShapeDtypeStruct((BATCH,), jnp.float32),
    mesh=plsc.VectorSubcoreMesh(core_axis_name="c", subcore_axis_name="s"),
    scratch_types=[
        pltpu.VMEM((CHUNKS_PER_W, CHUNK), jnp.int32),
        pltpu.VMEM((CHUNKS_PER_W, CHUNK), jnp.int32),
        pltpu.VMEM((CHUNKS_PER_W, CHUNK), jnp.int32),
        pltpu.VMEM((CHUNKS_PER_W, CHUNK), jnp.int32),
        pltpu.VMEM((CHUNK, PAD_DIM), jnp.float32),
        pltpu.VMEM((CHUNK, PAD_DIM), jnp.float32),
        pltpu.VMEM((B_PER_W,), jnp.float32),
        pltpu.VMEM((B_PER_W,), jnp.float32),
        pltpu.VMEM((B_PER_W,), jnp.float32),
        pltpu.VMEM((16,), jnp.float32),
        pltpu.SemaphoreType.DMA,
        pltpu.SemaphoreType.DMA,
    ],
    compiler_params=pltpu.CompilerParams(needs_layout_passes=False,
                                         use_tc_tiling_on_sc=True),
)
def _k2(*refs):
    _k2_body(*refs)


def kernel(user_idx, item_idx, user_factors, item_factors, user_bias,
           item_bias, global_bias):
    uidx = user_idx.astype(jnp.int32)
    iidx = item_idx.astype(jnp.int32)
    pos = lax.iota(jnp.int32, BATCH)
    su_r, su_b = lax.sort_key_val(uidx, pos)
    si_r, si_b = lax.sort_key_val(iidx, pos)
    inv_u = jnp.zeros((BATCH,), jnp.int32).at[su_b].set(pos)
    inv_i = jnp.zeros((BATCH,), jnp.int32).at[si_b].set(pos)
    shape2d = (BATCH // CHUNK, CHUNK)
    u_stage, v_stage = _k1(su_r.reshape(shape2d), si_r.reshape(shape2d),
                           user_factors.T, item_factors.T,
                           user_factors[TAIL_START:, :].T,
                           item_factors[TAIL_START:, :].T)
    gb = jnp.broadcast_to(global_bias, (16,))
    return _k2(uidx.reshape(shape2d), iidx.reshape(shape2d),
               inv_u.reshape(shape2d), inv_i.reshape(shape2d),
               u_stage, v_stage,
               user_bias.reshape(-1), item_bias.reshape(-1), gb)
